# baseline (speedup 1.0000x reference)
import jax
import jax.numpy as jnp
from jax.experimental import pallas as pl
from jax.experimental.pallas import tpu as pltpu

_TILE = 1024


def _moe_body(x_ref, wg_ref, wu_ref, wd_ref, o_ref):
    xb = x_ref[...].astype(jnp.bfloat16)

    logits = jax.lax.dot_general(
        xb, wg_ref[...].astype(jnp.bfloat16), (((1,), (1,)), ((), ())),
        preferred_element_type=jnp.float32)
    m1 = jnp.max(logits, axis=-1, keepdims=True)
    eb = jnp.exp((logits - m1).astype(jnp.bfloat16))
    denom = jnp.sum(eb, axis=-1, keepdims=True).astype(jnp.float32)
    is_max = logits == m1
    nmax = jnp.sum(is_max.astype(jnp.bfloat16), axis=-1, keepdims=True)
    e2m = jnp.max(jnp.where(is_max, jnp.bfloat16(0), eb), axis=-1,
                  keepdims=True)
    e2 = jnp.where(nmax > 1.5, jnp.float32(1), e2m.astype(jnp.float32))
    p = (1.0 + e2) / denom
    scale = (p / (p + 1e-9)).astype(jnp.bfloat16)

    up = jax.lax.dot_general(
        xb, wu_ref[...].astype(jnp.bfloat16), (((1,), (1,)), ((), ())),
        preferred_element_type=jnp.float32).astype(jnp.bfloat16)
    h = up * jax.nn.sigmoid(up) * scale
    o_ref[...] = jax.lax.dot_general(
        h, wd_ref[...].astype(jnp.bfloat16), (((1,), (1,)), ((), ())),
        preferred_element_type=jnp.float32)


@jax.jit
def kernel(x, W_gate, W_up, W_down):
    B_, S_, D_ = x.shape
    T = B_ * S_
    xf = x.reshape(T, D_)
    ne, ed = W_gate.shape[0], W_up.shape[0]

    grid = (T // _TILE,)
    out = pl.pallas_call(
        _moe_body,
        grid=grid,
        in_specs=[
            pl.BlockSpec((_TILE, D_), lambda i: (i, 0)),
            pl.BlockSpec((ne, D_), lambda i: (0, 0)),
            pl.BlockSpec((ed, D_), lambda i: (0, 0)),
            pl.BlockSpec((D_, ed), lambda i: (0, 0)),
        ],
        out_specs=pl.BlockSpec((_TILE, D_), lambda i: (i, 0)),
        out_shape=jax.ShapeDtypeStruct((T, D_), jnp.float32),
        compiler_params=pltpu.CompilerParams(
            dimension_semantics=("parallel",)),
    )(xf, W_gate, W_up, W_down)
    return out.reshape(B_, S_, D_)


# speedup vs baseline: 1.0196x; 1.0196x over previous
import jax
import jax.numpy as jnp
from jax.experimental import pallas as pl
from jax.experimental.pallas import tpu as pltpu

_TILE = 2048


def _moe_body(x_ref, wg_ref, wu_ref, wd_ref, o_ref):
    xb = x_ref[...].astype(jnp.bfloat16)

    logits = jax.lax.dot_general(
        xb, wg_ref[...].astype(jnp.bfloat16), (((1,), (1,)), ((), ())),
        preferred_element_type=jnp.float32)
    m1 = jnp.max(logits, axis=-1, keepdims=True)
    eb = jnp.exp((logits - m1).astype(jnp.bfloat16))
    denom = jnp.sum(eb, axis=-1, keepdims=True).astype(jnp.float32)
    is_max = logits == m1
    nmax = jnp.sum(is_max.astype(jnp.bfloat16), axis=-1, keepdims=True)
    e2m = jnp.max(jnp.where(is_max, jnp.bfloat16(0), eb), axis=-1,
                  keepdims=True)
    e2 = jnp.where(nmax > 1.5, jnp.float32(1), e2m.astype(jnp.float32))
    p = (1.0 + e2) / denom
    scale = (p / (p + 1e-9)).astype(jnp.bfloat16)

    up = jax.lax.dot_general(
        xb, wu_ref[...].astype(jnp.bfloat16), (((1,), (1,)), ((), ())),
        preferred_element_type=jnp.float32).astype(jnp.bfloat16)
    h = up * jax.nn.sigmoid(up) * scale
    o_ref[...] = jax.lax.dot_general(
        h, wd_ref[...].astype(jnp.bfloat16), (((1,), (1,)), ((), ())),
        preferred_element_type=jnp.float32)


@jax.jit
def kernel(x, W_gate, W_up, W_down):
    B_, S_, D_ = x.shape
    T = B_ * S_
    xf = x.reshape(T, D_)
    ne, ed = W_gate.shape[0], W_up.shape[0]

    grid = (T // _TILE,)
    out = pl.pallas_call(
        _moe_body,
        grid=grid,
        in_specs=[
            pl.BlockSpec((_TILE, D_), lambda i: (i, 0)),
            pl.BlockSpec((ne, D_), lambda i: (0, 0)),
            pl.BlockSpec((ed, D_), lambda i: (0, 0)),
            pl.BlockSpec((D_, ed), lambda i: (0, 0)),
        ],
        out_specs=pl.BlockSpec((_TILE, D_), lambda i: (i, 0)),
        out_shape=jax.ShapeDtypeStruct((T, D_), jnp.float32),
        compiler_params=pltpu.CompilerParams(
            dimension_semantics=("parallel",)),
    )(xf, W_gate, W_up, W_down)
    return out.reshape(B_, S_, D_)


# speedup vs baseline: 1.0291x; 1.0093x over previous
import functools

import jax
import jax.numpy as jnp
from jax.experimental import pallas as pl
from jax.experimental.pallas import tpu as pltpu

_TILE = 2048


def _moe_body(x_ref, wg_ref, wu_ref, wd_ref, o_ref):
    xt = x_ref[...]

    logits = jax.lax.dot_general(
        xt, wg_ref[...], (((1,), (1,)), ((), ())),
        preferred_element_type=jnp.float32)
    ne = logits.shape[-1]
    m1 = jnp.max(logits, axis=-1, keepdims=True)
    e = jnp.exp(logits - m1)
    denom = jnp.sum(e, axis=-1, keepdims=True)
    col = jax.lax.broadcasted_iota(jnp.int32, logits.shape, 1)
    is_max = logits == m1
    argmax1 = jnp.min(jnp.where(is_max, col, ne), axis=-1, keepdims=True)
    e2 = jnp.max(jnp.where(col == argmax1, 0.0, e), axis=-1, keepdims=True)
    p = (1.0 + e2) / denom
    scale = p / (p + 1e-9)

    xb = xt.astype(jnp.bfloat16)
    up = jax.lax.dot_general(
        xb, wu_ref[...].astype(jnp.bfloat16), (((1,), (1,)), ((), ())),
        preferred_element_type=jnp.float32)
    h = (up * jax.nn.sigmoid(up)).astype(jnp.bfloat16)
    out = jax.lax.dot_general(
        h, wd_ref[...].astype(jnp.bfloat16), (((1,), (1,)), ((), ())),
        preferred_element_type=jnp.float32)
    o_ref[...] = out * scale


@jax.jit
def kernel(x, W_gate, W_up, W_down):
    B_, S_, D_ = x.shape
    T = B_ * S_
    xf = x.reshape(T, D_)
    ne, ed = W_gate.shape[0], W_up.shape[0]

    grid = (T // _TILE,)
    out = pl.pallas_call(
        _moe_body,
        grid=grid,
        in_specs=[
            pl.BlockSpec((_TILE, D_), lambda i: (i, 0)),
            pl.BlockSpec((ne, D_), lambda i: (0, 0)),
            pl.BlockSpec((ed, D_), lambda i: (0, 0)),
            pl.BlockSpec((D_, ed), lambda i: (0, 0)),
        ],
        out_specs=pl.BlockSpec((_TILE, D_), lambda i: (i, 0)),
        out_shape=jax.ShapeDtypeStruct((T, D_), jnp.float32),
        compiler_params=pltpu.CompilerParams(
            dimension_semantics=("parallel",)),
    )(xf, W_gate, W_up, W_down)
    return out.reshape(B_, S_, D_)
